# Initial kernel scaffold; baseline (speedup 1.0000x reference)
import jax
import jax.numpy as jnp
from jax.experimental import pallas as pl


def kernel(x, edge_index, batch, W_conv, b_conv, W_lin, b_lin):
    raise NotImplementedError("write your pallas kernel here")


# speedup vs baseline: 10.9690x; 10.9690x over previous
import functools

import jax
import jax.numpy as jnp
from jax import lax
from jax.experimental import pallas as pl
from jax.experimental.pallas import tpu as pltpu
from jax.experimental.pallas import tpu_sc as plsc

N = 10000
NP = 10240
E = 320000
EP = 327680
D = 128
H = 128
G = 16
NC, NS = 2, 16
BK = 128
NB1 = EP // (NC * NS * BK)
NB3 = EP // (NS * BK)
RPT = NP // NS

_mesh2 = plsc.VectorSubcoreMesh(
    core_axis_name="c", subcore_axis_name="s", num_cores=NC, num_subcores=NS)
_mesh1 = plsc.VectorSubcoreMesh(
    core_axis_name="c", subcore_axis_name="s", num_cores=1, num_subcores=NS)


@functools.partial(
    pl.kernel,
    out_type=jax.ShapeDtypeStruct((NP,), jnp.float32),
    mesh=_mesh1,
    scratch_types=[
        pltpu.VMEM((NB3, BK), jnp.int32),
        pltpu.VMEM((BK,), jnp.float32),
        pltpu.VMEM((RPT,), jnp.float32),
        pltpu.VMEM_SHARED((NP,), jnp.float32),
    ],
)
def _deg_kernel(col_hbm, out_hbm, colv, ones_v, zeros_v, hist):
    sid = lax.axis_index("s")
    base = sid * RPT
    pltpu.sync_copy(col_hbm.at[sid], colv)
    zero16 = jnp.zeros((16,), jnp.float32)
    one16 = jnp.full((16,), 1.0, jnp.float32)
    for r in range(BK // 16):
        ones_v[pl.ds(r * 16, 16)] = one16
    for r in range(RPT // 16):
        zeros_v[pl.ds(r * 16, 16)] = zero16
    pltpu.sync_copy(zeros_v, hist.at[pl.ds(base, RPT)])
    plsc.subcore_barrier()

    def hbody(j, carry):
        pltpu.sync_copy(ones_v, hist.at[colv.at[j]], add=True)
        return carry

    lax.fori_loop(0, NB3, hbody, 0)
    plsc.subcore_barrier()
    pltpu.sync_copy(hist.at[pl.ds(base, RPT)], out_hbm.at[pl.ds(base, RPT)])


def _y_body(x_ref, w_ref, dp_ref, y_ref):
    deg = dp_ref[...] + 1.0
    dis = 1.0 / jnp.sqrt(deg)
    xw = jnp.dot(x_ref[...], w_ref[...], preferred_element_type=jnp.float32)
    y_ref[...] = dis * xw


_BLK = 1024


def _y_call(x_p, W_conv, deg2):
    return pl.pallas_call(
        _y_body,
        grid=(NP // _BLK,),
        in_specs=[
            pl.BlockSpec((_BLK, D), lambda i: (i, 0)),
            pl.BlockSpec((D, H), lambda i: (0, 0)),
            pl.BlockSpec((_BLK, 1), lambda i: (i, 0)),
        ],
        out_specs=pl.BlockSpec((_BLK, H), lambda i: (i, 0)),
        out_shape=jax.ShapeDtypeStruct((NP, H), jnp.float32),
    )(x_p, W_conv, deg2)


CHUNK = 32
NCHUNK = NB3 // CHUNK


@functools.partial(
    pl.kernel,
    out_type=jax.ShapeDtypeStruct((NP, H), jnp.float32),
    mesh=_mesh1,
    scratch_types=[
        pltpu.VMEM((CHUNK, BK), jnp.int32),
        pltpu.VMEM((CHUNK, BK), jnp.int32),
        pltpu.VMEM((BK, H), jnp.float32),
        pltpu.VMEM((BK, H), jnp.float32),
        pltpu.VMEM_SHARED((NP, H), jnp.float32),
        pltpu.SemaphoreType.DMA,
        pltpu.SemaphoreType.DMA,
    ],
)
def _gs_kernel(y_hbm, row_hbm, col_hbm, out_hbm,
               rowv, colv, b0, b1, acc, s0, s1):
    sid = lax.axis_index("s")
    base = sid * RPT
    pltpu.sync_copy(y_hbm.at[pl.ds(base, RPT)], acc.at[pl.ds(base, RPT)])
    plsc.subcore_barrier()

    bufs = (b0, b1)
    sems = (s0, s1)

    for c in range(NCHUNK):
        pltpu.sync_copy(row_hbm.at[sid, pl.ds(c * CHUNK, CHUNK)], rowv)
        pltpu.sync_copy(col_hbm.at[sid, pl.ds(c * CHUNK, CHUNK)], colv)
        pltpu.async_copy(y_hbm.at[rowv.at[0]], b0, s0)
        pltpu.async_copy(y_hbm.at[rowv.at[1]], b1, s1)

        def body(t, carry):
            for b in range(2):
                j = 2 * t + b
                pltpu.make_async_copy(y_hbm.at[pl.ds(0, BK)],
                                      bufs[b], sems[b]).wait()
                pltpu.sync_copy(bufs[b], acc.at[colv.at[j]], add=True)
                pltpu.async_copy(y_hbm.at[rowv.at[j + 2]], bufs[b], sems[b])
            return carry

        lax.fori_loop(0, CHUNK // 2 - 1, body, 0)
        for b in range(2):
            j = CHUNK - 2 + b
            pltpu.make_async_copy(y_hbm.at[pl.ds(0, BK)],
                                  bufs[b], sems[b]).wait()
            pltpu.sync_copy(bufs[b], acc.at[colv.at[j]], add=True)

    plsc.subcore_barrier()
    pltpu.sync_copy(acc.at[pl.ds(base, RPT)], out_hbm.at[pl.ds(base, RPT)])


def _pool_body(hp_ref, dp_ref, b_ref, bc_ref, wl_ref, bl_ref,
               out_ref, xpool_ref, pool_acc):
    i = pl.program_id(0)

    @pl.when(i == 0)
    def _():
        pool_acc[...] = jnp.full((G, H), -jnp.inf, jnp.float32)

    deg = dp_ref[...] + 1.0
    dis = 1.0 / jnp.sqrt(deg)
    h = dis * hp_ref[...] + bc_ref[...]
    h = jnp.where(h > 0, h, 0.01 * h)
    b2 = b_ref[0]
    neg = jnp.float32(-jnp.inf)
    rows = [jnp.max(jnp.where(b2 == g, h, neg), axis=0, keepdims=True)
            for g in range(G)]
    news = jnp.concatenate(rows, axis=0)
    pool_acc[...] = jnp.maximum(pool_acc[...], news)

    @pl.when(i == (NP // _BLK) - 1)
    def _():
        xp = pool_acc[...]
        xpool_ref[...] = xp
        out_ref[...] = (jnp.dot(xp, wl_ref[...],
                                preferred_element_type=jnp.float32)
                        + bl_ref[...])


def _pool_call(hp, deg2, batch3, b_conv2, W_lin, b_lin2):
    return pl.pallas_call(
        _pool_body,
        grid=(NP // _BLK,),
        in_specs=[
            pl.BlockSpec((_BLK, H), lambda i: (i, 0)),
            pl.BlockSpec((_BLK, 1), lambda i: (i, 0)),
            pl.BlockSpec((1, _BLK, 1), lambda i: (i, 0, 0)),
            pl.BlockSpec((1, H), lambda i: (0, 0)),
            pl.BlockSpec((H, 10), lambda i: (0, 0)),
            pl.BlockSpec((1, 10), lambda i: (0, 0)),
        ],
        out_specs=[
            pl.BlockSpec((G, 10), lambda i: (0, 0)),
            pl.BlockSpec((G, H), lambda i: (0, 0)),
        ],
        out_shape=[
            jax.ShapeDtypeStruct((G, 10), jnp.float32),
            jax.ShapeDtypeStruct((G, H), jnp.float32),
        ],
        scratch_shapes=[pltpu.VMEM((G, H), jnp.float32)],
    )(hp, deg2, batch3, b_conv2, W_lin, b_lin2)


def kernel(x, edge_index, batch, W_conv, b_conv, W_lin, b_lin):
    pad_idx = jnp.full((EP - E,), N, jnp.int32)
    row_p = jnp.concatenate([edge_index[0], pad_idx])
    col_p = jnp.concatenate([edge_index[1], pad_idx])
    x_p = jnp.pad(x, ((0, NP - N), (0, 0)))
    batch3 = jnp.pad(batch, (0, NP - N), constant_values=G).reshape(
        NP // _BLK, _BLK, 1)

    col3 = col_p.reshape(NS, NB3, BK)
    deg1 = _deg_kernel(col3).reshape(NP, 1)
    y = _y_call(x_p, W_conv, deg1)
    hp = _gs_kernel(y, row_p.reshape(NS, NB3, BK), col3)
    out, x_pool = _pool_call(hp, deg1, batch3, b_conv.reshape(1, H),
                             W_lin, b_lin.reshape(1, 10))
    return (out, x_pool)


# speedup vs baseline: 12.8059x; 1.1675x over previous
import functools

import jax
import jax.numpy as jnp
from jax import lax
from jax.experimental import pallas as pl
from jax.experimental.pallas import tpu as pltpu
from jax.experimental.pallas import tpu_sc as plsc

N = 10000
NP = 10240
E = 320000
EP = 327680
D = 128
H = 128
G = 16
NC, NS = 2, 16
BK = 128
NB1 = EP // (NC * NS * BK)
NB3 = EP // (NS * BK)
RPT = NP // NS

_mesh2 = plsc.VectorSubcoreMesh(
    core_axis_name="c", subcore_axis_name="s", num_cores=NC, num_subcores=NS)
_mesh1 = plsc.VectorSubcoreMesh(
    core_axis_name="c", subcore_axis_name="s", num_cores=1, num_subcores=NS)


@functools.partial(
    pl.kernel,
    out_type=jax.ShapeDtypeStruct((NP,), jnp.float32),
    mesh=_mesh1,
    scratch_types=[
        pltpu.VMEM((NB3, BK), jnp.int32),
        pltpu.VMEM((BK,), jnp.float32),
        pltpu.VMEM((RPT,), jnp.float32),
        pltpu.VMEM_SHARED((NP,), jnp.float32),
    ],
)
def _deg_kernel(col_hbm, out_hbm, colv, ones_v, zeros_v, hist):
    sid = lax.axis_index("s")
    base = sid * RPT
    pltpu.sync_copy(col_hbm.at[sid], colv)
    zero16 = jnp.zeros((16,), jnp.float32)
    one16 = jnp.full((16,), 1.0, jnp.float32)
    for r in range(BK // 16):
        ones_v[pl.ds(r * 16, 16)] = one16
    for r in range(RPT // 16):
        zeros_v[pl.ds(r * 16, 16)] = zero16
    pltpu.sync_copy(zeros_v, hist.at[pl.ds(base, RPT)])
    plsc.subcore_barrier()

    def hbody(j, carry):
        pltpu.sync_copy(ones_v, hist.at[colv.at[j]], add=True)
        return carry

    lax.fori_loop(0, NB3, hbody, 0)
    plsc.subcore_barrier()
    pltpu.sync_copy(hist.at[pl.ds(base, RPT)], out_hbm.at[pl.ds(base, RPT)])


def _y_body(x_ref, w_ref, dp_ref, y_ref):
    deg = dp_ref[...] + 1.0
    dis = 1.0 / jnp.sqrt(deg)
    xw = jnp.dot(x_ref[...], w_ref[...], preferred_element_type=jnp.float32)
    y_ref[...] = dis * xw


_BLK = 1024


def _y_call(x_p, W_conv, deg2):
    return pl.pallas_call(
        _y_body,
        grid=(NP // _BLK,),
        in_specs=[
            pl.BlockSpec((_BLK, D), lambda i: (i, 0)),
            pl.BlockSpec((D, H), lambda i: (0, 0)),
            pl.BlockSpec((_BLK, 1), lambda i: (i, 0)),
        ],
        out_specs=pl.BlockSpec((_BLK, H), lambda i: (i, 0)),
        out_shape=jax.ShapeDtypeStruct((NP, H), jnp.float32),
    )(x_p, W_conv, deg2)


CHUNK = 16
NCHUNK = NB1 // CHUNK


@functools.partial(
    pl.kernel,
    out_type=jax.ShapeDtypeStruct((NC, NP, H), jnp.float32),
    mesh=_mesh2,
    scratch_types=[
        pltpu.VMEM((CHUNK, BK), jnp.int32),
        pltpu.VMEM((CHUNK, BK), jnp.int32),
        pltpu.VMEM((BK, H), jnp.float32),
        pltpu.VMEM((BK, H), jnp.float32),
        pltpu.VMEM_SHARED((NP, H), jnp.float32),
        pltpu.SemaphoreType.DMA,
        pltpu.SemaphoreType.DMA,
    ],
)
def _gs_kernel(y_hbm, row_hbm, col_hbm, out_hbm,
               rowv, colv, b0, b1, acc, s0, s1):
    cid = lax.axis_index("c")
    sid = lax.axis_index("s")
    wid = cid * NS + sid
    base = sid * RPT
    pltpu.sync_copy(y_hbm.at[pl.ds(base, RPT)], acc.at[pl.ds(base, RPT)])
    plsc.subcore_barrier()

    bufs = (b0, b1)
    sems = (s0, s1)

    for c in range(NCHUNK):
        pltpu.sync_copy(row_hbm.at[wid, pl.ds(c * CHUNK, CHUNK)], rowv)
        pltpu.sync_copy(col_hbm.at[wid, pl.ds(c * CHUNK, CHUNK)], colv)
        pltpu.async_copy(y_hbm.at[rowv.at[0]], b0, s0)
        pltpu.async_copy(y_hbm.at[rowv.at[1]], b1, s1)

        def body(t, carry):
            for b in range(2):
                j = 2 * t + b
                pltpu.make_async_copy(y_hbm.at[pl.ds(0, BK)],
                                      bufs[b], sems[b]).wait()
                pltpu.sync_copy(bufs[b], acc.at[colv.at[j]], add=True)
                pltpu.async_copy(y_hbm.at[rowv.at[j + 2]], bufs[b], sems[b])
            return carry

        lax.fori_loop(0, CHUNK // 2 - 1, body, 0)
        for b in range(2):
            j = CHUNK - 2 + b
            pltpu.make_async_copy(y_hbm.at[pl.ds(0, BK)],
                                  bufs[b], sems[b]).wait()
            pltpu.sync_copy(bufs[b], acc.at[colv.at[j]], add=True)

    plsc.subcore_barrier()
    pltpu.sync_copy(acc.at[pl.ds(base, RPT)],
                    out_hbm.at[cid, pl.ds(base, RPT)])


def _pool_body(hp_ref, y_ref, dp_ref, b_ref, bc_ref, wl_ref, bl_ref,
               out_ref, xpool_ref, pool_acc):
    i = pl.program_id(0)

    @pl.when(i == 0)
    def _():
        pool_acc[...] = jnp.full((G, H), -jnp.inf, jnp.float32)

    deg = dp_ref[...] + 1.0
    dis = 1.0 / jnp.sqrt(deg)
    h = dis * (hp_ref[0] + hp_ref[1] - y_ref[...]) + bc_ref[...]
    h = jnp.where(h > 0, h, 0.01 * h)
    b2 = b_ref[0]
    neg = jnp.float32(-jnp.inf)
    rows = [jnp.max(jnp.where(b2 == g, h, neg), axis=0, keepdims=True)
            for g in range(G)]
    news = jnp.concatenate(rows, axis=0)
    pool_acc[...] = jnp.maximum(pool_acc[...], news)

    @pl.when(i == (NP // _BLK) - 1)
    def _():
        xp = pool_acc[...]
        xpool_ref[...] = xp
        out_ref[...] = (jnp.dot(xp, wl_ref[...],
                                preferred_element_type=jnp.float32)
                        + bl_ref[...])


def _pool_call(hp, y, deg2, batch3, b_conv2, W_lin, b_lin2):
    return pl.pallas_call(
        _pool_body,
        grid=(NP // _BLK,),
        in_specs=[
            pl.BlockSpec((NC, _BLK, H), lambda i: (0, i, 0)),
            pl.BlockSpec((_BLK, H), lambda i: (i, 0)),
            pl.BlockSpec((_BLK, 1), lambda i: (i, 0)),
            pl.BlockSpec((1, _BLK, 1), lambda i: (i, 0, 0)),
            pl.BlockSpec((1, H), lambda i: (0, 0)),
            pl.BlockSpec((H, 10), lambda i: (0, 0)),
            pl.BlockSpec((1, 10), lambda i: (0, 0)),
        ],
        out_specs=[
            pl.BlockSpec((G, 10), lambda i: (0, 0)),
            pl.BlockSpec((G, H), lambda i: (0, 0)),
        ],
        out_shape=[
            jax.ShapeDtypeStruct((G, 10), jnp.float32),
            jax.ShapeDtypeStruct((G, H), jnp.float32),
        ],
        scratch_shapes=[pltpu.VMEM((G, H), jnp.float32)],
    )(hp, y, deg2, batch3, b_conv2, W_lin, b_lin2)


def kernel(x, edge_index, batch, W_conv, b_conv, W_lin, b_lin):
    pad_idx = jnp.full((EP - E,), N, jnp.int32)
    row_p = jnp.concatenate([edge_index[0], pad_idx])
    col_p = jnp.concatenate([edge_index[1], pad_idx])
    x_p = jnp.pad(x, ((0, NP - N), (0, 0)))
    batch3 = jnp.pad(batch, (0, NP - N), constant_values=G).reshape(
        NP // _BLK, _BLK, 1)

    deg1 = _deg_kernel(col_p.reshape(NS, NB3, BK)).reshape(NP, 1)
    y = _y_call(x_p, W_conv, deg1)
    hp = _gs_kernel(y, row_p.reshape(NC * NS, NB1, BK),
                    col_p.reshape(NC * NS, NB1, BK))
    out, x_pool = _pool_call(hp, y, deg1, batch3, b_conv.reshape(1, H),
                             W_lin, b_lin.reshape(1, 10))
    return (out, x_pool)


# speedup vs baseline: 36.8359x; 2.8765x over previous
import functools

import jax
import jax.numpy as jnp
from jax import lax
from jax.experimental import pallas as pl
from jax.experimental.pallas import tpu as pltpu
from jax.experimental.pallas import tpu_sc as plsc

N = 10000
NP = 10240
E = 320000
EP = 327680
D = 128
H = 128
G = 16
NC, NS = 2, 16
BK = 128
NB1 = EP // (NC * NS * BK)
NB3 = EP // (NS * BK)
RPT = NP // NS

_mesh2 = plsc.VectorSubcoreMesh(
    core_axis_name="c", subcore_axis_name="s", num_cores=NC, num_subcores=NS)
_mesh1 = plsc.VectorSubcoreMesh(
    core_axis_name="c", subcore_axis_name="s", num_cores=1, num_subcores=NS)


@functools.partial(
    pl.kernel,
    out_type=jax.ShapeDtypeStruct((NP,), jnp.float32),
    mesh=_mesh1,
    scratch_types=[
        pltpu.VMEM((NB3, BK), jnp.int32),
        pltpu.VMEM((BK,), jnp.float32),
        pltpu.VMEM((RPT,), jnp.float32),
        pltpu.VMEM_SHARED((NP,), jnp.float32),
    ],
)
def _deg_kernel(col_hbm, out_hbm, colv, ones_v, zeros_v, hist):
    sid = lax.axis_index("s")
    base = sid * RPT
    pltpu.sync_copy(col_hbm.at[sid], colv)
    zero16 = jnp.zeros((16,), jnp.float32)
    one16 = jnp.full((16,), 1.0, jnp.float32)
    for r in range(BK // 16):
        ones_v[pl.ds(r * 16, 16)] = one16
    for r in range(RPT // 16):
        zeros_v[pl.ds(r * 16, 16)] = zero16
    pltpu.sync_copy(zeros_v, hist.at[pl.ds(base, RPT)])
    plsc.subcore_barrier()

    def hbody(j, carry):
        pltpu.sync_copy(ones_v, hist.at[colv.at[j]], add=True)
        return carry

    lax.fori_loop(0, NB3, hbody, 0)
    plsc.subcore_barrier()
    pltpu.sync_copy(hist.at[pl.ds(base, RPT)], out_hbm.at[pl.ds(base, RPT)])


def _y_body(x_ref, w_ref, dp_ref, y_ref):
    deg = dp_ref[...] + 1.0
    dis = 1.0 / jnp.sqrt(deg)
    xw = jnp.dot(x_ref[...], w_ref[...], preferred_element_type=jnp.float32)
    y_ref[...] = dis * xw


_BLK = 1024


def _y_call(x_p, W_conv, deg2):
    return pl.pallas_call(
        _y_body,
        grid=(NP // _BLK,),
        in_specs=[
            pl.BlockSpec((_BLK, D), lambda i: (i, 0)),
            pl.BlockSpec((D, H), lambda i: (0, 0)),
            pl.BlockSpec((_BLK, 1), lambda i: (i, 0)),
        ],
        out_specs=pl.BlockSpec((_BLK, H), lambda i: (i, 0)),
        out_shape=jax.ShapeDtypeStruct((NP, H), jnp.float32),
    )(x_p, W_conv, deg2)


CHUNK = 16
NCHUNK = NB1 // CHUNK


@functools.partial(
    pl.kernel,
    out_type=jax.ShapeDtypeStruct((NC, NP, H), jnp.float32),
    mesh=_mesh2,
    scratch_types=[
        pltpu.VMEM((CHUNK, BK), jnp.int32),
        pltpu.VMEM((CHUNK, BK), jnp.int32),
        pltpu.VMEM((BK, H), jnp.float32),
        pltpu.VMEM((BK, H), jnp.float32),
        pltpu.VMEM_SHARED((NP, H), jnp.float32),
        pltpu.SemaphoreType.DMA,
        pltpu.SemaphoreType.DMA,
    ],
)
def _gs_kernel(y_hbm, row_hbm, col_hbm, out_hbm,
               rowv, colv, b0, b1, acc, s0, s1):
    cid = lax.axis_index("c")
    sid = lax.axis_index("s")
    wid = cid * NS + sid
    base = sid * RPT
    pltpu.sync_copy(y_hbm.at[pl.ds(base, RPT)], acc.at[pl.ds(base, RPT)])
    plsc.subcore_barrier()

    bufs = (b0, b1)
    sems = (s0, s1)

    for c in range(NCHUNK):
        pltpu.sync_copy(row_hbm.at[wid, pl.ds(c * CHUNK, CHUNK)], rowv)
        pltpu.sync_copy(col_hbm.at[wid, pl.ds(c * CHUNK, CHUNK)], colv)
        pltpu.async_copy(y_hbm.at[rowv.at[0]], b0, s0)
        pltpu.async_copy(y_hbm.at[rowv.at[1]], b1, s1)

        def body(t, carry):
            for b in range(2):
                j = 2 * t + b
                pltpu.make_async_copy(y_hbm.at[pl.ds(0, BK)],
                                      bufs[b], sems[b]).wait()
                pltpu.sync_copy(bufs[b], acc.at[colv.at[j]], add=True)
                pltpu.async_copy(y_hbm.at[rowv.at[j + 2]], bufs[b], sems[b])
            return carry

        lax.fori_loop(0, CHUNK // 2 - 1, body, 0)
        for b in range(2):
            j = CHUNK - 2 + b
            pltpu.make_async_copy(y_hbm.at[pl.ds(0, BK)],
                                  bufs[b], sems[b]).wait()
            pltpu.sync_copy(bufs[b], acc.at[colv.at[j]], add=True)

    plsc.subcore_barrier()
    pltpu.sync_copy(acc.at[pl.ds(base, RPT)],
                    out_hbm.at[cid, pl.ds(base, RPT)])


def _pool_body(hp_ref, y_ref, dp_ref, b_ref, bc_ref, wl_ref, bl_ref,
               out_ref, xpool_ref, pool_acc):
    i = pl.program_id(0)

    @pl.when(i == 0)
    def _():
        pool_acc[...] = jnp.full((G, H), -jnp.inf, jnp.float32)

    deg = dp_ref[...] + 1.0
    dis = 1.0 / jnp.sqrt(deg)
    h = dis * (hp_ref[0] + hp_ref[1] - y_ref[...]) + bc_ref[...]
    h = jnp.where(h > 0, h, 0.01 * h)
    b2 = b_ref[0]
    neg = jnp.float32(-jnp.inf)
    rows = [jnp.max(jnp.where(b2 == g, h, neg), axis=0, keepdims=True)
            for g in range(G)]
    news = jnp.concatenate(rows, axis=0)
    pool_acc[...] = jnp.maximum(pool_acc[...], news)

    @pl.when(i == (NP // _BLK) - 1)
    def _():
        xp = pool_acc[...]
        xpool_ref[...] = xp
        out_ref[...] = (jnp.dot(xp, wl_ref[...],
                                preferred_element_type=jnp.float32)
                        + bl_ref[...])


def _pool_call(hp, y, deg2, batch3, b_conv2, W_lin, b_lin2):
    return pl.pallas_call(
        _pool_body,
        grid=(NP // _BLK,),
        in_specs=[
            pl.BlockSpec((NC, _BLK, H), lambda i: (0, i, 0)),
            pl.BlockSpec((_BLK, H), lambda i: (i, 0)),
            pl.BlockSpec((_BLK, 1), lambda i: (i, 0)),
            pl.BlockSpec((1, _BLK, 1), lambda i: (i, 0, 0)),
            pl.BlockSpec((1, H), lambda i: (0, 0)),
            pl.BlockSpec((H, 10), lambda i: (0, 0)),
            pl.BlockSpec((1, 10), lambda i: (0, 0)),
        ],
        out_specs=[
            pl.BlockSpec((G, 10), lambda i: (0, 0)),
            pl.BlockSpec((G, H), lambda i: (0, 0)),
        ],
        out_shape=[
            jax.ShapeDtypeStruct((G, 10), jnp.float32),
            jax.ShapeDtypeStruct((G, H), jnp.float32),
        ],
        scratch_shapes=[pltpu.VMEM((G, H), jnp.float32)],
    )(hp, y, deg2, batch3, b_conv2, W_lin, b_lin2)


def kernel(x, edge_index, batch, W_conv, b_conv, W_lin, b_lin):
    pad_idx = N + (jnp.arange(EP - E, dtype=jnp.int32) % (NP - N))
    row_p = jnp.concatenate([edge_index[0], pad_idx])
    col_p = jnp.concatenate([edge_index[1], pad_idx])
    x_p = jnp.pad(x, ((0, NP - N), (0, 0)))
    batch3 = jnp.pad(batch, (0, NP - N), constant_values=G).reshape(
        NP // _BLK, _BLK, 1)

    deg1 = _deg_kernel(col_p.reshape(NS, NB3, BK)).reshape(NP, 1)
    y = _y_call(x_p, W_conv, deg1)
    hp = _gs_kernel(y, row_p.reshape(NC * NS, NB1, BK),
                    col_p.reshape(NC * NS, NB1, BK))
    out, x_pool = _pool_call(hp, y, deg1, batch3, b_conv.reshape(1, H),
                             W_lin, b_lin.reshape(1, 10))
    return (out, x_pool)


# speedup vs baseline: 38.6392x; 1.0490x over previous
import functools

import jax
import jax.numpy as jnp
from jax import lax
from jax.experimental import pallas as pl
from jax.experimental.pallas import tpu as pltpu
from jax.experimental.pallas import tpu_sc as plsc

N = 10000
NP = 10240
E = 320000
EP = 327680
D = 128
H = 128
G = 16
NC, NS = 2, 16
BK = 128
NB1 = EP // (NC * NS * BK)
NB3 = EP // (NS * BK)
RPT = NP // NS

_mesh2 = plsc.VectorSubcoreMesh(
    core_axis_name="c", subcore_axis_name="s", num_cores=NC, num_subcores=NS)
_mesh1 = plsc.VectorSubcoreMesh(
    core_axis_name="c", subcore_axis_name="s", num_cores=1, num_subcores=NS)


@functools.partial(
    pl.kernel,
    out_type=jax.ShapeDtypeStruct((NP,), jnp.float32),
    mesh=_mesh1,
    scratch_types=[
        pltpu.VMEM((NB3, BK), jnp.int32),
        pltpu.VMEM((BK,), jnp.float32),
        pltpu.VMEM((RPT,), jnp.float32),
        pltpu.VMEM_SHARED((NP,), jnp.float32),
    ],
)
def _deg_kernel(col_hbm, out_hbm, colv, ones_v, zeros_v, hist):
    sid = lax.axis_index("s")
    base = sid * RPT
    pltpu.sync_copy(col_hbm.at[sid], colv)
    zero16 = jnp.zeros((16,), jnp.float32)
    one16 = jnp.full((16,), 1.0, jnp.float32)
    for r in range(BK // 16):
        ones_v[pl.ds(r * 16, 16)] = one16
    for r in range(RPT // 16):
        zeros_v[pl.ds(r * 16, 16)] = zero16
    pltpu.sync_copy(zeros_v, hist.at[pl.ds(base, RPT)])
    plsc.subcore_barrier()

    def hbody(j, carry):
        pltpu.sync_copy(ones_v, hist.at[colv.at[j]], add=True)
        return carry

    lax.fori_loop(0, NB3, hbody, 0)
    plsc.subcore_barrier()
    pltpu.sync_copy(hist.at[pl.ds(base, RPT)], out_hbm.at[pl.ds(base, RPT)])


_BLK = 1024


def _xw_body(x_ref, w_ref, xw_ref):
    xw_ref[...] = jnp.dot(x_ref[...], w_ref[...],
                          preferred_element_type=jnp.float32)


def _xw_call(x_p, W_conv):
    return pl.pallas_call(
        _xw_body,
        grid=(NP // _BLK,),
        in_specs=[
            pl.BlockSpec((_BLK, D), lambda i: (i, 0)),
            pl.BlockSpec((D, H), lambda i: (0, 0)),
        ],
        out_specs=pl.BlockSpec((_BLK, H), lambda i: (i, 0)),
        out_shape=jax.ShapeDtypeStruct((NP, H), jnp.float32),
    )(x_p, W_conv)


def _y_body(xw_ref, dp_ref, y_ref):
    deg = dp_ref[...] + 1.0
    dis = 1.0 / jnp.sqrt(deg)
    y_ref[...] = dis * xw_ref[...]


def _y_call(xw, deg2):
    return pl.pallas_call(
        _y_body,
        grid=(NP // _BLK,),
        in_specs=[
            pl.BlockSpec((_BLK, H), lambda i: (i, 0)),
            pl.BlockSpec((_BLK, 1), lambda i: (i, 0)),
        ],
        out_specs=pl.BlockSpec((_BLK, H), lambda i: (i, 0)),
        out_shape=jax.ShapeDtypeStruct((NP, H), jnp.float32),
    )(xw, deg2)


CHUNK = 40
NCHUNK = NB1 // CHUNK


@functools.partial(
    pl.kernel,
    out_type=jax.ShapeDtypeStruct((NC, NP, H), jnp.float32),
    mesh=_mesh2,
    scratch_types=[
        pltpu.VMEM((CHUNK, BK), jnp.int32),
        pltpu.VMEM((CHUNK, BK), jnp.int32),
        pltpu.VMEM((BK, H), jnp.float32),
        pltpu.VMEM((BK, H), jnp.float32),
        pltpu.VMEM_SHARED((NP, H), jnp.float32),
        pltpu.SemaphoreType.DMA,
        pltpu.SemaphoreType.DMA,
    ],
)
def _gs_kernel(y_hbm, row_hbm, col_hbm, out_hbm,
               rowv, colv, b0, b1, acc, s0, s1):
    cid = lax.axis_index("c")
    sid = lax.axis_index("s")
    wid = cid * NS + sid
    base = sid * RPT
    pltpu.sync_copy(y_hbm.at[pl.ds(base, RPT)], acc.at[pl.ds(base, RPT)])
    plsc.subcore_barrier()

    bufs = (b0, b1)
    sems = (s0, s1)

    for c in range(NCHUNK):
        pltpu.sync_copy(row_hbm.at[wid, pl.ds(c * CHUNK, CHUNK)], rowv)
        pltpu.sync_copy(col_hbm.at[wid, pl.ds(c * CHUNK, CHUNK)], colv)
        pltpu.async_copy(y_hbm.at[rowv.at[0]], b0, s0)
        pltpu.async_copy(y_hbm.at[rowv.at[1]], b1, s1)

        def body(t, carry):
            for b in range(2):
                j = 2 * t + b
                pltpu.make_async_copy(y_hbm.at[pl.ds(0, BK)],
                                      bufs[b], sems[b]).wait()
                pltpu.sync_copy(bufs[b], acc.at[colv.at[j]], add=True)
                pltpu.async_copy(y_hbm.at[rowv.at[j + 2]], bufs[b], sems[b])
            return carry

        lax.fori_loop(0, CHUNK // 2 - 1, body, 0)
        for b in range(2):
            j = CHUNK - 2 + b
            pltpu.make_async_copy(y_hbm.at[pl.ds(0, BK)],
                                  bufs[b], sems[b]).wait()
            pltpu.sync_copy(bufs[b], acc.at[colv.at[j]], add=True)

    plsc.subcore_barrier()
    pltpu.sync_copy(acc.at[pl.ds(base, RPT)],
                    out_hbm.at[cid, pl.ds(base, RPT)])


def _pool_body(hp_ref, y_ref, dp_ref, b_ref, bc_ref, wl_ref, bl_ref,
               out_ref, xpool_ref, pool_acc):
    i = pl.program_id(0)

    @pl.when(i == 0)
    def _():
        pool_acc[...] = jnp.full((G, H), -jnp.inf, jnp.float32)

    deg = dp_ref[...] + 1.0
    dis = 1.0 / jnp.sqrt(deg)
    h = dis * (hp_ref[0] + hp_ref[1] - y_ref[...]) + bc_ref[...]
    h = jnp.where(h > 0, h, 0.01 * h)
    b2 = b_ref[0]
    neg = jnp.float32(-jnp.inf)
    rows = [jnp.max(jnp.where(b2 == g, h, neg), axis=0, keepdims=True)
            for g in range(G)]
    news = jnp.concatenate(rows, axis=0)
    pool_acc[...] = jnp.maximum(pool_acc[...], news)

    @pl.when(i == (NP // _BLK) - 1)
    def _():
        xp = pool_acc[...]
        xpool_ref[...] = xp
        out_ref[...] = (jnp.dot(xp, wl_ref[...],
                                preferred_element_type=jnp.float32)
                        + bl_ref[...])


def _pool_call(hp, y, deg2, batch3, b_conv2, W_lin, b_lin2):
    return pl.pallas_call(
        _pool_body,
        grid=(NP // _BLK,),
        in_specs=[
            pl.BlockSpec((NC, _BLK, H), lambda i: (0, i, 0)),
            pl.BlockSpec((_BLK, H), lambda i: (i, 0)),
            pl.BlockSpec((_BLK, 1), lambda i: (i, 0)),
            pl.BlockSpec((1, _BLK, 1), lambda i: (i, 0, 0)),
            pl.BlockSpec((1, H), lambda i: (0, 0)),
            pl.BlockSpec((H, 10), lambda i: (0, 0)),
            pl.BlockSpec((1, 10), lambda i: (0, 0)),
        ],
        out_specs=[
            pl.BlockSpec((G, 10), lambda i: (0, 0)),
            pl.BlockSpec((G, H), lambda i: (0, 0)),
        ],
        out_shape=[
            jax.ShapeDtypeStruct((G, 10), jnp.float32),
            jax.ShapeDtypeStruct((G, H), jnp.float32),
        ],
        scratch_shapes=[pltpu.VMEM((G, H), jnp.float32)],
    )(hp, y, deg2, batch3, b_conv2, W_lin, b_lin2)


def kernel(x, edge_index, batch, W_conv, b_conv, W_lin, b_lin):
    pad_idx = N + (jnp.arange(EP - E, dtype=jnp.int32) % (NP - N))
    row_p = jnp.concatenate([edge_index[0], pad_idx])
    col_p = jnp.concatenate([edge_index[1], pad_idx])
    x_p = jnp.pad(x, ((0, NP - N), (0, 0)))
    batch3 = jnp.pad(batch, (0, NP - N), constant_values=G).reshape(
        NP // _BLK, _BLK, 1)

    deg1 = _deg_kernel(col_p.reshape(NS, NB3, BK)).reshape(NP, 1)
    xw = _xw_call(x_p, W_conv)
    y = _y_call(xw, deg1)
    hp = _gs_kernel(y, row_p.reshape(NC * NS, NB1, BK),
                    col_p.reshape(NC * NS, NB1, BK))
    out, x_pool = _pool_call(hp, y, deg1, batch3, b_conv.reshape(1, H),
                             W_lin, b_lin.reshape(1, 10))
    return (out, x_pool)


# speedup vs baseline: 40.7793x; 1.0554x over previous
import functools

import jax
import jax.numpy as jnp
from jax import lax
from jax.experimental import pallas as pl
from jax.experimental.pallas import tpu as pltpu
from jax.experimental.pallas import tpu_sc as plsc

N = 10000
NP = 10240
E = 320000
EP = 327680
D = 128
H = 128
G = 16
NC, NS = 2, 16
BK = 128
NB1 = EP // (NC * NS * BK)
NB3 = EP // (NS * BK)
RPT = NP // NS

_mesh2 = plsc.VectorSubcoreMesh(
    core_axis_name="c", subcore_axis_name="s", num_cores=NC, num_subcores=NS)
_mesh1 = plsc.VectorSubcoreMesh(
    core_axis_name="c", subcore_axis_name="s", num_cores=1, num_subcores=NS)


@functools.partial(
    pl.kernel,
    out_type=jax.ShapeDtypeStruct((NP,), jnp.float32),
    mesh=_mesh1,
    scratch_types=[
        pltpu.VMEM((NB3, BK), jnp.int32),
        pltpu.VMEM((BK,), jnp.float32),
        pltpu.VMEM((RPT,), jnp.float32),
        pltpu.VMEM_SHARED((NP,), jnp.float32),
    ],
)
def _deg_kernel(col_hbm, out_hbm, colv, ones_v, zeros_v, hist):
    sid = lax.axis_index("s")
    base = sid * RPT
    pltpu.sync_copy(col_hbm.at[sid], colv)
    zero16 = jnp.zeros((16,), jnp.float32)
    one16 = jnp.full((16,), 1.0, jnp.float32)
    for r in range(BK // 16):
        ones_v[pl.ds(r * 16, 16)] = one16
    for r in range(RPT // 16):
        zeros_v[pl.ds(r * 16, 16)] = zero16
    pltpu.sync_copy(zeros_v, hist.at[pl.ds(base, RPT)])
    plsc.subcore_barrier()

    def hbody(j, carry):
        pltpu.sync_copy(ones_v, hist.at[colv.at[j]], add=True)
        return carry

    lax.fori_loop(0, NB3, hbody, 0)
    plsc.subcore_barrier()
    pltpu.sync_copy(hist.at[pl.ds(base, RPT)], out_hbm.at[pl.ds(base, RPT)])


_BLK = 1024


def _y_body(x_ref, w_ref, dp_ref, y_ref):
    deg = dp_ref[...] + 1.0
    dis = 1.0 / jnp.sqrt(deg)
    xw = jnp.dot(x_ref[...], w_ref[...], preferred_element_type=jnp.float32)
    y_ref[...] = dis * xw


def _y_call(x_p, W_conv, deg2):
    return pl.pallas_call(
        _y_body,
        grid=(NP // _BLK,),
        in_specs=[
            pl.BlockSpec((_BLK, D), lambda i: (i, 0)),
            pl.BlockSpec((D, H), lambda i: (0, 0)),
            pl.BlockSpec((_BLK, 1), lambda i: (i, 0)),
        ],
        out_specs=pl.BlockSpec((_BLK, H), lambda i: (i, 0)),
        out_shape=jax.ShapeDtypeStruct((NP, H), jnp.float32),
    )(x_p, W_conv, deg2)


CHUNK = 40
NCHUNK = NB1 // CHUNK


@functools.partial(
    pl.kernel,
    out_type=jax.ShapeDtypeStruct((NC, NP, H), jnp.float32),
    mesh=_mesh2,
    scratch_types=[
        pltpu.VMEM((CHUNK, BK), jnp.int32),
        pltpu.VMEM((CHUNK, BK), jnp.int32),
        pltpu.VMEM((BK, H), jnp.float32),
        pltpu.VMEM((BK, H), jnp.float32),
        pltpu.VMEM_SHARED((NP, H), jnp.float32),
        pltpu.SemaphoreType.DMA,
        pltpu.SemaphoreType.DMA,
    ],
)
def _gs_kernel(y_hbm, row_hbm, col_hbm, out_hbm,
               rowv, colv, b0, b1, acc, s0, s1):
    cid = lax.axis_index("c")
    sid = lax.axis_index("s")
    wid = cid * NS + sid
    base = sid * RPT
    pltpu.sync_copy(y_hbm.at[pl.ds(base, RPT)], acc.at[pl.ds(base, RPT)])
    plsc.subcore_barrier()

    bufs = (b0, b1)
    sems = (s0, s1)

    for c in range(NCHUNK):
        pltpu.sync_copy(row_hbm.at[wid, pl.ds(c * CHUNK, CHUNK)], rowv)
        pltpu.sync_copy(col_hbm.at[wid, pl.ds(c * CHUNK, CHUNK)], colv)
        pltpu.async_copy(y_hbm.at[rowv.at[0]], b0, s0)
        pltpu.async_copy(y_hbm.at[rowv.at[1]], b1, s1)

        def body(t, carry):
            for b in range(2):
                j = 2 * t + b
                pltpu.make_async_copy(y_hbm.at[pl.ds(0, BK)],
                                      bufs[b], sems[b]).wait()
                pltpu.sync_copy(bufs[b], acc.at[colv.at[j]], add=True)
                pltpu.async_copy(y_hbm.at[rowv.at[j + 2]], bufs[b], sems[b])
            return carry

        lax.fori_loop(0, CHUNK // 2 - 1, body, 0)
        for b in range(2):
            j = CHUNK - 2 + b
            pltpu.make_async_copy(y_hbm.at[pl.ds(0, BK)],
                                  bufs[b], sems[b]).wait()
            pltpu.sync_copy(bufs[b], acc.at[colv.at[j]], add=True)

    plsc.subcore_barrier()
    pltpu.sync_copy(acc.at[pl.ds(base, RPT)],
                    out_hbm.at[cid, pl.ds(base, RPT)])


def _pool_body(hp_ref, y_ref, dp_ref, b_ref, bc_ref, wl_ref, bl_ref,
               out_ref, xpool_ref, pool_acc):
    i = pl.program_id(0)

    @pl.when(i == 0)
    def _():
        pool_acc[...] = jnp.full((G + 1, H), -jnp.inf, jnp.float32)

    deg = dp_ref[...] + 1.0
    dis = 1.0 / jnp.sqrt(deg)
    h = dis * (hp_ref[0] + hp_ref[1] - y_ref[...]) + bc_ref[...]
    h = jnp.where(h > 0, h, 0.01 * h)
    b2 = b_ref[0]
    neg = jnp.float32(-jnp.inf)
    gmin = jnp.min(b2)
    gmax = jnp.max(b2)

    def pool_one(g, carry):
        m2 = jnp.max(jnp.where(b2 == g, h, neg), axis=0, keepdims=True)
        cur = pool_acc[pl.ds(g, 1), :]
        pool_acc[pl.ds(g, 1), :] = jnp.maximum(cur, m2)
        return carry

    lax.fori_loop(gmin, gmax + 1, pool_one, 0)

    @pl.when(i == (NP // _BLK) - 1)
    def _():
        xp = pool_acc[pl.ds(0, G), :]
        xpool_ref[...] = xp
        out_ref[...] = (jnp.dot(xp, wl_ref[...],
                                preferred_element_type=jnp.float32)
                        + bl_ref[...])


def _pool_call(hp, y, deg2, batch3, b_conv2, W_lin, b_lin2):
    return pl.pallas_call(
        _pool_body,
        grid=(NP // _BLK,),
        in_specs=[
            pl.BlockSpec((NC, _BLK, H), lambda i: (0, i, 0)),
            pl.BlockSpec((_BLK, H), lambda i: (i, 0)),
            pl.BlockSpec((_BLK, 1), lambda i: (i, 0)),
            pl.BlockSpec((1, _BLK, 1), lambda i: (i, 0, 0)),
            pl.BlockSpec((1, H), lambda i: (0, 0)),
            pl.BlockSpec((H, 10), lambda i: (0, 0)),
            pl.BlockSpec((1, 10), lambda i: (0, 0)),
        ],
        out_specs=[
            pl.BlockSpec((G, 10), lambda i: (0, 0)),
            pl.BlockSpec((G, H), lambda i: (0, 0)),
        ],
        out_shape=[
            jax.ShapeDtypeStruct((G, 10), jnp.float32),
            jax.ShapeDtypeStruct((G, H), jnp.float32),
        ],
        scratch_shapes=[pltpu.VMEM((G + 1, H), jnp.float32)],
    )(hp, y, deg2, batch3, b_conv2, W_lin, b_lin2)


def kernel(x, edge_index, batch, W_conv, b_conv, W_lin, b_lin):
    pad_idx = N + (jnp.arange(EP - E, dtype=jnp.int32) % (NP - N))
    row_p = jnp.concatenate([edge_index[0], pad_idx])
    col_p = jnp.concatenate([edge_index[1], pad_idx])
    x_p = jnp.pad(x, ((0, NP - N), (0, 0)))
    batch3 = jnp.pad(batch, (0, NP - N), constant_values=G).reshape(
        NP // _BLK, _BLK, 1)

    deg1 = _deg_kernel(col_p.reshape(NS, NB3, BK)).reshape(NP, 1)
    y = _y_call(x_p, W_conv, deg1)
    hp = _gs_kernel(y, row_p.reshape(NC * NS, NB1, BK),
                    col_p.reshape(NC * NS, NB1, BK))
    out, x_pool = _pool_call(hp, y, deg1, batch3, b_conv.reshape(1, H),
                             W_lin, b_lin.reshape(1, 10))
    return (out, x_pool)


# speedup vs baseline: 42.0501x; 1.0312x over previous
import functools

import jax
import jax.numpy as jnp
from jax import lax
from jax.experimental import pallas as pl
from jax.experimental.pallas import tpu as pltpu
from jax.experimental.pallas import tpu_sc as plsc

N = 10000
NP = 10240
E = 320000
EP = 327680
D = 128
H = 128
G = 16
NC, NS = 2, 16
BK = 128
NB1 = EP // (NC * NS * BK)
NB3 = EP // (NS * BK)
RPT = NP // NS

_mesh2 = plsc.VectorSubcoreMesh(
    core_axis_name="c", subcore_axis_name="s", num_cores=NC, num_subcores=NS)
_mesh1 = plsc.VectorSubcoreMesh(
    core_axis_name="c", subcore_axis_name="s", num_cores=1, num_subcores=NS)


@functools.partial(
    pl.kernel,
    out_type=jax.ShapeDtypeStruct((NC, NP), jnp.float32),
    mesh=_mesh2,
    scratch_types=[
        pltpu.VMEM((NB1, BK), jnp.int32),
        pltpu.VMEM((BK,), jnp.float32),
        pltpu.VMEM((RPT,), jnp.float32),
        pltpu.VMEM_SHARED((NP,), jnp.float32),
    ],
)
def _deg_kernel(col_hbm, out_hbm, colv, ones_v, zeros_v, hist):
    cid = lax.axis_index("c")
    sid = lax.axis_index("s")
    wid = cid * NS + sid
    base = sid * RPT
    pltpu.sync_copy(col_hbm.at[wid], colv)
    zero16 = jnp.zeros((16,), jnp.float32)
    one16 = jnp.full((16,), 1.0, jnp.float32)
    for r in range(BK // 16):
        ones_v[pl.ds(r * 16, 16)] = one16
    for r in range(RPT // 16):
        zeros_v[pl.ds(r * 16, 16)] = zero16
    pltpu.sync_copy(zeros_v, hist.at[pl.ds(base, RPT)])
    plsc.subcore_barrier()

    def hbody(j, carry):
        pltpu.sync_copy(ones_v, hist.at[colv.at[j]], add=True)
        return carry

    lax.fori_loop(0, NB1, hbody, 0)
    plsc.subcore_barrier()
    pltpu.sync_copy(hist.at[pl.ds(base, RPT)],
                    out_hbm.at[cid, pl.ds(base, RPT)])


_BLK = 1024


def _y_body(x_ref, w_ref, dp_ref, y_ref):
    deg = dp_ref[0] + dp_ref[1] + 1.0
    dis = 1.0 / jnp.sqrt(deg)
    xw = jnp.dot(x_ref[...], w_ref[...], preferred_element_type=jnp.float32)
    y_ref[...] = dis * xw


def _y_call(x_p, W_conv, deg2):
    return pl.pallas_call(
        _y_body,
        grid=(NP // _BLK,),
        in_specs=[
            pl.BlockSpec((_BLK, D), lambda i: (i, 0)),
            pl.BlockSpec((D, H), lambda i: (0, 0)),
            pl.BlockSpec((NC, _BLK, 1), lambda i: (0, i, 0)),
        ],
        out_specs=pl.BlockSpec((_BLK, H), lambda i: (i, 0)),
        out_shape=jax.ShapeDtypeStruct((NP, H), jnp.float32),
    )(x_p, W_conv, deg2)


CHUNK = 40
NCHUNK = NB1 // CHUNK


@functools.partial(
    pl.kernel,
    out_type=jax.ShapeDtypeStruct((NC, NP, H), jnp.float32),
    mesh=_mesh2,
    scratch_types=[
        pltpu.VMEM((CHUNK, BK), jnp.int32),
        pltpu.VMEM((CHUNK, BK), jnp.int32),
        pltpu.VMEM((BK, H), jnp.float32),
        pltpu.VMEM((BK, H), jnp.float32),
        pltpu.VMEM_SHARED((NP, H), jnp.float32),
        pltpu.SemaphoreType.DMA,
        pltpu.SemaphoreType.DMA,
    ],
)
def _gs_kernel(y_hbm, row_hbm, col_hbm, out_hbm,
               rowv, colv, b0, b1, acc, s0, s1):
    cid = lax.axis_index("c")
    sid = lax.axis_index("s")
    wid = cid * NS + sid
    base = sid * RPT
    bufs = (b0, b1)
    sems = (s0, s1)

    pltpu.sync_copy(row_hbm.at[wid, pl.ds(0, CHUNK)], rowv)
    pltpu.sync_copy(col_hbm.at[wid, pl.ds(0, CHUNK)], colv)
    pltpu.async_copy(y_hbm.at[rowv.at[0]], b0, s0)
    pltpu.async_copy(y_hbm.at[rowv.at[1]], b1, s1)
    pltpu.sync_copy(y_hbm.at[pl.ds(base, RPT)], acc.at[pl.ds(base, RPT)])
    plsc.subcore_barrier()

    for c in range(NCHUNK):
        if c > 0:
            pltpu.sync_copy(row_hbm.at[wid, pl.ds(c * CHUNK, CHUNK)], rowv)
            pltpu.sync_copy(col_hbm.at[wid, pl.ds(c * CHUNK, CHUNK)], colv)
            pltpu.async_copy(y_hbm.at[rowv.at[0]], b0, s0)
            pltpu.async_copy(y_hbm.at[rowv.at[1]], b1, s1)

        def body(t, carry):
            for b in range(2):
                j = 2 * t + b
                pltpu.make_async_copy(y_hbm.at[pl.ds(0, BK)],
                                      bufs[b], sems[b]).wait()
                pltpu.sync_copy(bufs[b], acc.at[colv.at[j]], add=True)
                pltpu.async_copy(y_hbm.at[rowv.at[j + 2]], bufs[b], sems[b])
            return carry

        lax.fori_loop(0, CHUNK // 2 - 1, body, 0)
        for b in range(2):
            j = CHUNK - 2 + b
            pltpu.make_async_copy(y_hbm.at[pl.ds(0, BK)],
                                  bufs[b], sems[b]).wait()
            pltpu.sync_copy(bufs[b], acc.at[colv.at[j]], add=True)

    plsc.subcore_barrier()
    pltpu.sync_copy(acc.at[pl.ds(base, RPT)],
                    out_hbm.at[cid, pl.ds(base, RPT)])


def _pool_body(hp_ref, y_ref, dp_ref, b_ref, bc_ref, wl_ref, bl_ref,
               out_ref, xpool_ref, pool_acc):
    i = pl.program_id(0)

    @pl.when(i == 0)
    def _():
        pool_acc[...] = jnp.full((G + 1, H), -jnp.inf, jnp.float32)

    deg = dp_ref[0] + dp_ref[1] + 1.0
    dis = 1.0 / jnp.sqrt(deg)
    h = dis * (hp_ref[0] + hp_ref[1] - y_ref[...]) + bc_ref[...]
    h = jnp.where(h > 0, h, 0.01 * h)
    b2 = b_ref[0]
    neg = jnp.float32(-jnp.inf)
    gmin = jnp.min(b2)
    gmax = jnp.max(b2)

    def pool_one(g, carry):
        m2 = jnp.max(jnp.where(b2 == g, h, neg), axis=0, keepdims=True)
        cur = pool_acc[pl.ds(g, 1), :]
        pool_acc[pl.ds(g, 1), :] = jnp.maximum(cur, m2)
        return carry

    lax.fori_loop(gmin, gmax + 1, pool_one, 0)

    @pl.when(i == (NP // _BLK) - 1)
    def _():
        xp = pool_acc[pl.ds(0, G), :]
        xpool_ref[...] = xp
        out_ref[...] = (jnp.dot(xp, wl_ref[...],
                                preferred_element_type=jnp.float32)
                        + bl_ref[...])


def _pool_call(hp, y, deg2, batch3, b_conv2, W_lin, b_lin2):
    return pl.pallas_call(
        _pool_body,
        grid=(NP // _BLK,),
        in_specs=[
            pl.BlockSpec((NC, _BLK, H), lambda i: (0, i, 0)),
            pl.BlockSpec((_BLK, H), lambda i: (i, 0)),
            pl.BlockSpec((NC, _BLK, 1), lambda i: (0, i, 0)),
            pl.BlockSpec((1, _BLK, 1), lambda i: (i, 0, 0)),
            pl.BlockSpec((1, H), lambda i: (0, 0)),
            pl.BlockSpec((H, 10), lambda i: (0, 0)),
            pl.BlockSpec((1, 10), lambda i: (0, 0)),
        ],
        out_specs=[
            pl.BlockSpec((G, 10), lambda i: (0, 0)),
            pl.BlockSpec((G, H), lambda i: (0, 0)),
        ],
        out_shape=[
            jax.ShapeDtypeStruct((G, 10), jnp.float32),
            jax.ShapeDtypeStruct((G, H), jnp.float32),
        ],
        scratch_shapes=[pltpu.VMEM((G + 1, H), jnp.float32)],
    )(hp, y, deg2, batch3, b_conv2, W_lin, b_lin2)


def kernel(x, edge_index, batch, W_conv, b_conv, W_lin, b_lin):
    pad_idx = N + (jnp.arange(EP - E, dtype=jnp.int32) % (NP - N))
    row_p = jnp.concatenate([edge_index[0], pad_idx])
    col_p = jnp.concatenate([edge_index[1], pad_idx])
    x_p = jnp.pad(x, ((0, NP - N), (0, 0)))
    batch3 = jnp.pad(batch, (0, NP - N), constant_values=G).reshape(
        NP // _BLK, _BLK, 1)

    row3 = row_p.reshape(NC * NS, NB1, BK)
    col3 = col_p.reshape(NC * NS, NB1, BK)
    deg1 = _deg_kernel(col3).reshape(NC, NP, 1)
    y = _y_call(x_p, W_conv, deg1)
    hp = _gs_kernel(y, row3, col3)
    out, x_pool = _pool_call(hp, y, deg1, batch3, b_conv.reshape(1, H),
                             W_lin, b_lin.reshape(1, 10))
    return (out, x_pool)


# speedup vs baseline: 42.5480x; 1.0118x over previous
import functools

import jax
import jax.numpy as jnp
from jax import lax
from jax.experimental import pallas as pl
from jax.experimental.pallas import tpu as pltpu
from jax.experimental.pallas import tpu_sc as plsc

N = 10000
NP = 10240
E = 320000
EP = 327680
D = 128
H = 128
G = 16
NC, NS = 2, 16
BK = 128
NB1 = EP // (NC * NS * BK)
NB3 = EP // (NS * BK)
RPT = NP // NS

_mesh2 = plsc.VectorSubcoreMesh(
    core_axis_name="c", subcore_axis_name="s", num_cores=NC, num_subcores=NS)
_mesh1 = plsc.VectorSubcoreMesh(
    core_axis_name="c", subcore_axis_name="s", num_cores=1, num_subcores=NS)


@functools.partial(
    pl.kernel,
    out_type=jax.ShapeDtypeStruct((NC, NP), jnp.float32),
    mesh=_mesh2,
    scratch_types=[
        pltpu.VMEM((NB1, BK), jnp.int32),
        pltpu.VMEM((BK,), jnp.float32),
        pltpu.VMEM((RPT,), jnp.float32),
        pltpu.VMEM_SHARED((NP,), jnp.float32),
    ],
)
def _deg_kernel(col_hbm, out_hbm, colv, ones_v, zeros_v, hist):
    cid = lax.axis_index("c")
    sid = lax.axis_index("s")
    wid = cid * NS + sid
    base = sid * RPT
    pltpu.sync_copy(col_hbm.at[wid], colv)
    zero16 = jnp.zeros((16,), jnp.float32)
    one16 = jnp.full((16,), 1.0, jnp.float32)
    for r in range(BK // 16):
        ones_v[pl.ds(r * 16, 16)] = one16
    for r in range(RPT // 16):
        zeros_v[pl.ds(r * 16, 16)] = zero16
    pltpu.sync_copy(zeros_v, hist.at[pl.ds(base, RPT)])
    plsc.subcore_barrier()

    def hbody(j, carry):
        pltpu.sync_copy(ones_v, hist.at[colv.at[j]], add=True)
        return carry

    lax.fori_loop(0, NB1, hbody, 0)
    plsc.subcore_barrier()
    pltpu.sync_copy(hist.at[pl.ds(base, RPT)],
                    out_hbm.at[cid, pl.ds(base, RPT)])


_BLK = 1024


def _y_body(x_ref, w_ref, dp_ref, y_ref):
    deg = dp_ref[0] + dp_ref[1] + 1.0
    dis = 1.0 / jnp.sqrt(deg)
    xw = jnp.dot(x_ref[...], w_ref[...], preferred_element_type=jnp.float32)
    y_ref[...] = dis * xw


def _y_call(x_p, W_conv, deg2):
    return pl.pallas_call(
        _y_body,
        grid=(NP // _BLK,),
        in_specs=[
            pl.BlockSpec((_BLK, D), lambda i: (i, 0)),
            pl.BlockSpec((D, H), lambda i: (0, 0)),
            pl.BlockSpec((NC, _BLK, 1), lambda i: (0, i, 0)),
        ],
        out_specs=pl.BlockSpec((_BLK, H), lambda i: (i, 0)),
        out_shape=jax.ShapeDtypeStruct((NP, H), jnp.float32),
    )(x_p, W_conv, deg2)


CCHUNK = 40


@functools.partial(
    pl.kernel,
    out_type=jax.ShapeDtypeStruct((NC, NP, H), jnp.float32),
    mesh=_mesh2,
    scratch_types=[
        pltpu.VMEM((NB1, BK), jnp.int32),
        pltpu.VMEM((CCHUNK, BK), jnp.int32),
        pltpu.VMEM((BK, H), jnp.float32),
        pltpu.VMEM((BK, H), jnp.float32),
        pltpu.VMEM_SHARED((NP, H), jnp.float32),
        pltpu.SemaphoreType.DMA,
        pltpu.SemaphoreType.DMA,
    ],
)
def _gs_kernel(y_hbm, row_hbm, col_hbm, out_hbm,
               rowv, colv, b0, b1, acc, s0, s1):
    cid = lax.axis_index("c")
    sid = lax.axis_index("s")
    wid = cid * NS + sid
    base = sid * RPT
    bufs = (b0, b1)
    sems = (s0, s1)

    pltpu.sync_copy(row_hbm.at[wid], rowv)
    pltpu.sync_copy(col_hbm.at[wid, pl.ds(0, CCHUNK)], colv)
    pltpu.async_copy(y_hbm.at[rowv.at[0]], b0, s0)
    pltpu.async_copy(y_hbm.at[rowv.at[1]], b1, s1)
    pltpu.sync_copy(y_hbm.at[pl.ds(base, RPT)], acc.at[pl.ds(base, RPT)])
    plsc.subcore_barrier()

    def body(t, carry):
        @pl.when(t == CCHUNK // 2)
        def _():
            pltpu.sync_copy(col_hbm.at[wid, pl.ds(CCHUNK, CCHUNK)], colv)

        for b in range(2):
            j = 2 * t + b
            cj = jnp.where(j >= CCHUNK, j - CCHUNK, j)
            pltpu.make_async_copy(y_hbm.at[pl.ds(0, BK)],
                                  bufs[b], sems[b]).wait()
            pltpu.sync_copy(bufs[b], acc.at[colv.at[cj]], add=True)
            pltpu.async_copy(y_hbm.at[rowv.at[j + 2]], bufs[b], sems[b])
        return carry

    lax.fori_loop(0, NB1 // 2 - 1, body, 0)
    for b in range(2):
        j = NB1 - 2 + b
        pltpu.make_async_copy(y_hbm.at[pl.ds(0, BK)],
                              bufs[b], sems[b]).wait()
        pltpu.sync_copy(bufs[b], acc.at[colv.at[j - CCHUNK]], add=True)

    plsc.subcore_barrier()
    pltpu.sync_copy(acc.at[pl.ds(base, RPT)],
                    out_hbm.at[cid, pl.ds(base, RPT)])


def _pool_body(hp_ref, y_ref, dp_ref, b_ref, bc_ref, wl_ref, bl_ref,
               out_ref, xpool_ref, pool_acc):
    i = pl.program_id(0)

    @pl.when(i == 0)
    def _():
        pool_acc[...] = jnp.full((G + 1, H), -jnp.inf, jnp.float32)

    deg = dp_ref[0] + dp_ref[1] + 1.0
    dis = 1.0 / jnp.sqrt(deg)
    h = dis * (hp_ref[0] + hp_ref[1] - y_ref[...]) + bc_ref[...]
    h = jnp.where(h > 0, h, 0.01 * h)
    b2 = b_ref[0]
    neg = jnp.float32(-jnp.inf)
    gmin = jnp.min(b2)
    gmax = jnp.max(b2)

    def pool_one(g, carry):
        m2 = jnp.max(jnp.where(b2 == g, h, neg), axis=0, keepdims=True)
        cur = pool_acc[pl.ds(g, 1), :]
        pool_acc[pl.ds(g, 1), :] = jnp.maximum(cur, m2)
        return carry

    lax.fori_loop(gmin, gmax + 1, pool_one, 0)

    @pl.when(i == (NP // _BLK) - 1)
    def _():
        xp = pool_acc[pl.ds(0, G), :]
        xpool_ref[...] = xp
        out_ref[...] = (jnp.dot(xp, wl_ref[...],
                                preferred_element_type=jnp.float32)
                        + bl_ref[...])


def _pool_call(hp, y, deg2, batch3, b_conv2, W_lin, b_lin2):
    return pl.pallas_call(
        _pool_body,
        grid=(NP // _BLK,),
        in_specs=[
            pl.BlockSpec((NC, _BLK, H), lambda i: (0, i, 0)),
            pl.BlockSpec((_BLK, H), lambda i: (i, 0)),
            pl.BlockSpec((NC, _BLK, 1), lambda i: (0, i, 0)),
            pl.BlockSpec((1, _BLK, 1), lambda i: (i, 0, 0)),
            pl.BlockSpec((1, H), lambda i: (0, 0)),
            pl.BlockSpec((H, 10), lambda i: (0, 0)),
            pl.BlockSpec((1, 10), lambda i: (0, 0)),
        ],
        out_specs=[
            pl.BlockSpec((G, 10), lambda i: (0, 0)),
            pl.BlockSpec((G, H), lambda i: (0, 0)),
        ],
        out_shape=[
            jax.ShapeDtypeStruct((G, 10), jnp.float32),
            jax.ShapeDtypeStruct((G, H), jnp.float32),
        ],
        scratch_shapes=[pltpu.VMEM((G + 1, H), jnp.float32)],
    )(hp, y, deg2, batch3, b_conv2, W_lin, b_lin2)


def kernel(x, edge_index, batch, W_conv, b_conv, W_lin, b_lin):
    pad_idx = N + (jnp.arange(EP - E, dtype=jnp.int32) % (NP - N))
    row_p = jnp.concatenate([edge_index[0], pad_idx])
    col_p = jnp.concatenate([edge_index[1], pad_idx])
    x_p = jnp.pad(x, ((0, NP - N), (0, 0)))
    batch3 = jnp.pad(batch, (0, NP - N), constant_values=G).reshape(
        NP // _BLK, _BLK, 1)

    row3 = row_p.reshape(NC * NS, NB1, BK)
    col3 = col_p.reshape(NC * NS, NB1, BK)
    deg1 = _deg_kernel(col3).reshape(NC, NP, 1)
    y = _y_call(x_p, W_conv, deg1)
    hp = _gs_kernel(y, row3, col3)
    out, x_pool = _pool_call(hp, y, deg1, batch3, b_conv.reshape(1, H),
                             W_lin, b_lin.reshape(1, 10))
    return (out, x_pool)
